# baseline (speedup 1.0000x reference)
import functools

import numpy as np
import jax
import jax.numpy as jnp
from jax import lax
from jax.experimental import pallas as pl
from jax.experimental.pallas import tpu as pltpu
from jax.experimental.pallas import tpu_sc as plsc

_MAX_LEN = 4096
_NC, _NS, _L = 2, 16, 16
_NW = _NC * _NS
_C = 16
_NBUF = 4


def _factor_tables(d_feature):
    h = d_feature // 2
    scale_factor = -np.log(10000.0) / (h - 1)
    div_term = np.exp(np.arange(0, h) * scale_factor)
    tab = np.empty((48, d_feature), dtype=np.float32)
    for block, mult in ((0, 256.0), (1, 16.0), (2, 1.0)):
        ang = (mult * np.arange(16))[:, None] * div_term[None, :]
        tab[16 * block:16 * (block + 1), :h] = np.sin(ang)
        tab[16 * block:16 * (block + 1), h:] = np.cos(ang)
    return jnp.asarray(tab)


def _sc_body(n_rows, d, x_hbm, pos_hbm, tab_hbm, out_hbm,
             tab_v, idx_raw, in0, in1, in2, in3,
             sem_in0, sem_in1, sem_in2, sem_in3,
             sem_out0, sem_out1, sem_out2, sem_out3):
    wid = lax.axis_index("s") * _NC + lax.axis_index("c")
    rows_per_w = n_rows // _NW
    base0 = wid * rows_per_w
    n_chunks = rows_per_w // _C
    n_wins = n_chunks // _NBUF
    h = d // 2
    n_grp = h // _L

    bufs = (in0, in1, in2, in3)
    sem_in = (sem_in0, sem_in1, sem_in2, sem_in3)
    sem_out = (sem_out0, sem_out1, sem_out2, sem_out3)

    pltpu.sync_copy(tab_hbm, tab_v)
    pltpu.sync_copy(pos_hbm.at[pl.ds(base0, rows_per_w)], idx_raw)

    def issue_in(c, j):
        return pltpu.async_copy(
            x_hbm.at[pl.ds(base0 + c * _C, _C)], bufs[j], sem_in[j])

    def wait_in(j):
        pltpu.make_async_copy(
            x_hbm.at[pl.ds(base0, _C)], bufs[j], sem_in[j]).wait()

    def wait_out(j):
        pltpu.make_async_copy(
            bufs[j], out_hbm.at[pl.ds(base0, _C)], sem_out[j]).wait()

    def compute(c, j):
        iv = bufs[j]
        idx_vec = idx_raw[pl.ds(pl.multiple_of(c * _C, _L), _L)]
        rs = []
        for t in range(_C):
            p = idx_vec[t]
            rs.append((lax.shift_right_logical(p, 8),
                       (lax.shift_right_logical(p, 4) & 15) + 16,
                       (p & 15) + 32))

        @plsc.parallel_loop(0, n_grp, 1)
        def _(k):
            off = pl.multiple_of(lax.shift_left(k, 4), _L)
            offh = pl.multiple_of(off + h, _L)
            for t in range(_C):
                rx, ry, rz = rs[t]
                sx = tab_v[rx, pl.ds(off, _L)]
                cx = tab_v[rx, pl.ds(offh, _L)]
                sy = tab_v[ry, pl.ds(off, _L)]
                cy = tab_v[ry, pl.ds(offh, _L)]
                sz = tab_v[rz, pl.ds(off, _L)]
                cz = tab_v[rz, pl.ds(offh, _L)]
                syz = sy * cz + cy * sz
                czy = cy * cz - sy * sz
                iv[t, pl.ds(off, _L)] = (
                    iv[t, pl.ds(off, _L)] + sx * czy + cx * syz)
                iv[t, pl.ds(offh, _L)] = (
                    iv[t, pl.ds(offh, _L)] + cx * czy - sx * syz)

        pltpu.async_copy(iv, out_hbm.at[pl.ds(base0 + c * _C, _C)],
                         sem_out[j])

    issue_in(0, 0)
    issue_in(1, 1)

    def window(i, carry):
        for j in range(_NBUF):
            c = i * _NBUF + j
            wait_in(j)
            compute(c, j)
            jn = (j + 2) % _NBUF
            if j < 2:
                @pl.when(i > 0)
                def _():
                    wait_out(jn)
                    issue_in(c + 2, jn)

                @pl.when(i == 0)
                def _():
                    issue_in(c + 2, jn)
            else:
                @pl.when(i < n_wins - 1)
                def _():
                    wait_out(jn)
                    issue_in(c + 2, jn)

                @pl.when(i == n_wins - 1)
                def _():
                    wait_out(jn)
        return carry

    lax.fori_loop(0, n_wins, window, 0)
    wait_out(2)
    wait_out(3)


def _make_sc_call(n_rows, d):
    mesh = plsc.VectorSubcoreMesh(
        core_axis_name="c", subcore_axis_name="s",
        num_cores=_NC, num_subcores=_NS)
    rows_per_w = n_rows // _NW
    return pl.kernel(
        functools.partial(_sc_body, n_rows, d),
        out_type=jax.ShapeDtypeStruct((n_rows, d), jnp.float32),
        mesh=mesh,
        scratch_types=[
            pltpu.VMEM((48, d), jnp.float32),
            pltpu.VMEM((rows_per_w,), jnp.int32),
            pltpu.VMEM((_C, d), jnp.float32),
            pltpu.VMEM((_C, d), jnp.float32),
            pltpu.VMEM((_C, d), jnp.float32),
            pltpu.VMEM((_C, d), jnp.float32),
            pltpu.SemaphoreType.DMA,
            pltpu.SemaphoreType.DMA,
            pltpu.SemaphoreType.DMA,
            pltpu.SemaphoreType.DMA,
            pltpu.SemaphoreType.DMA,
            pltpu.SemaphoreType.DMA,
            pltpu.SemaphoreType.DMA,
            pltpu.SemaphoreType.DMA,
        ],
    )


def kernel(inputs, inputs_positions):
    b, t, d = inputs.shape
    n_rows = b * t
    x = inputs.reshape(n_rows, d)
    pos = inputs_positions.reshape(n_rows).astype(jnp.int32)
    tab = _factor_tables(d)
    out = _make_sc_call(n_rows, d)(x, pos, tab)
    return out.reshape(b, t, d)
